# baseline (speedup 1.0000x reference)
import functools

import jax
import jax.numpy as jnp
from jax import lax
from jax.experimental import pallas as pl
from jax.experimental.pallas import tpu as pltpu
from jax.experimental.pallas import tpu_sc as plsc

_N = 10000
_E = 320000
_D = 128
_CP = 16

_NC = 2
_NS = 16
_NW = _NC * _NS
_EW = _E // _NW
_C = 80
_NCH = _EW // _C

_RW = _N // _NS
_RC = 25
_NRC = _RW // _RC

_NCK = 5
_ECK = _E // _NCK
_EWK = _ECK // _NW
_NCHK = _EWK // _C

_TE = 2000
_GE = _ECK // _TE
_TN = 1000
_GN = _N // _TN

_f32 = jnp.float32


def _silu(x):
    return x * (1.0 / (1.0 + jnp.exp(-x)))


def _gather_body(h_hbm, cp_hbm, row_hbm, col_hbm,
                 hr_o, hc_o, cr_o, cc_o,
                 ir, ic,
                 hrA, hcA, crA, ccA, hrB, hcB, crB, ccB,
                 sgA, sgB, swA, swB):
    cid = lax.axis_index("c")
    sid = lax.axis_index("s")
    base = (sid * _NC + cid) * _EWK

    pltpu.sync_copy(row_hbm.at[pl.ds(base, _EWK)], ir)
    pltpu.sync_copy(col_hbm.at[pl.ds(base, _EWK)], ic)

    def gstart(k, hr_b, hc_b, cr_b, cc_b, sem):
        s = pl.ds(k * _C, _C)
        pltpu.async_copy(h_hbm.at[ir.at[s]], hr_b, sem)
        pltpu.async_copy(h_hbm.at[ic.at[s]], hc_b, sem)
        pltpu.async_copy(cp_hbm.at[ir.at[s]], cr_b, sem)
        pltpu.async_copy(cp_hbm.at[ic.at[s]], cc_b, sem)

    def gwait(hr_b, hc_b, cr_b, cc_b, sem):
        pltpu.make_async_copy(h_hbm.at[pl.ds(0, _C)], hr_b, sem).wait()
        pltpu.make_async_copy(h_hbm.at[pl.ds(0, _C)], hc_b, sem).wait()
        pltpu.make_async_copy(cp_hbm.at[pl.ds(0, _C)], cr_b, sem).wait()
        pltpu.make_async_copy(cp_hbm.at[pl.ds(0, _C)], cc_b, sem).wait()

    def wstart(k, hr_b, hc_b, cr_b, cc_b, sem):
        off = base + k * _C
        pltpu.async_copy(hr_b, hr_o.at[pl.ds(off, _C)], sem)
        pltpu.async_copy(hc_b, hc_o.at[pl.ds(off, _C)], sem)
        pltpu.async_copy(cr_b, cr_o.at[pl.ds(off, _C)], sem)
        pltpu.async_copy(cc_b, cc_o.at[pl.ds(off, _C)], sem)

    def wwait(hr_b, hc_b, cr_b, cc_b, sem):
        pltpu.make_async_copy(hr_b, hr_o.at[pl.ds(0, _C)], sem).wait()
        pltpu.make_async_copy(hc_b, hc_o.at[pl.ds(0, _C)], sem).wait()
        pltpu.make_async_copy(cr_b, cr_o.at[pl.ds(0, _C)], sem).wait()
        pltpu.make_async_copy(cc_b, cc_o.at[pl.ds(0, _C)], sem).wait()

    A = (hrA, hcA, crA, ccA)
    B = (hrB, hcB, crB, ccB)
    gstart(0, *A, sgA)

    def body(k2, carry):
        kA = 2 * k2
        kB = kA + 1

        @pl.when(kB < _NCHK)
        def _():
            gstart(kB, *B, sgB)

        gwait(*A, sgA)
        wstart(kA, *A, swA)

        @pl.when(kB < _NCHK)
        def _():
            gwait(*B, sgB)
            wstart(kB, *B, swB)

        wwait(*A, swA)

        @pl.when(kA + 2 < _NCHK)
        def _():
            gstart(kA + 2, *A, sgA)

        @pl.when(kB < _NCHK)
        def _():
            wwait(*B, swB)

        return carry

    lax.fori_loop(0, (_NCHK + 1) // 2, body, 0)


def _gather(h, cp, row, col):
    mesh = plsc.VectorSubcoreMesh(core_axis_name="c", subcore_axis_name="s",
                                  num_cores=_NC, num_subcores=_NS)
    hdt = h.dtype
    out_type = (jax.ShapeDtypeStruct((_ECK, _D), hdt),
                jax.ShapeDtypeStruct((_ECK, _D), hdt),
                jax.ShapeDtypeStruct((_ECK, _CP), _f32),
                jax.ShapeDtypeStruct((_ECK, _CP), _f32))
    scratch = [pltpu.VMEM((_EWK,), jnp.int32),
               pltpu.VMEM((_EWK,), jnp.int32),
               pltpu.VMEM((_C, _D), hdt), pltpu.VMEM((_C, _D), hdt),
               pltpu.VMEM((_C, _CP), _f32), pltpu.VMEM((_C, _CP), _f32),
               pltpu.VMEM((_C, _D), hdt), pltpu.VMEM((_C, _D), hdt),
               pltpu.VMEM((_C, _CP), _f32), pltpu.VMEM((_C, _CP), _f32),
               pltpu.SemaphoreType.DMA, pltpu.SemaphoreType.DMA,
               pltpu.SemaphoreType.DMA, pltpu.SemaphoreType.DMA]
    return pl.kernel(_gather_body, out_type=out_type, mesh=mesh,
                     scratch_types=scratch,
                     compiler_params=pltpu.CompilerParams(
                         use_tc_tiling_on_sc=False))(h, cp, row, col)


def _scatter_body(*args):
    efs = args[0:_NCK]
    trs = args[_NCK:2 * _NCK]
    cols = args[2 * _NCK:3 * _NCK]
    agg_o, cs_o = args[3 * _NCK], args[3 * _NCK + 1]
    (idxA, efA, trA, idxB, efB, trB,
     zbuf, zcb, agg_sh, cs_sh, slA, slB) = args[3 * _NCK + 2:]

    cid = lax.axis_index("c")
    sid = lax.axis_index("s")
    base = (sid * _NC + cid) * _EWK
    z16 = jnp.zeros((16,), _f32)

    def zrow(i, carry):
        for j in range(_D // 16):
            zbuf[i, pl.ds(j * 16, 16)] = z16
        zcb[i, :] = z16
        return carry

    lax.fori_loop(0, _RC, zrow, 0)
    r0 = sid * _RW
    for k in range(_NRC):
        pltpu.sync_copy(zbuf, agg_sh.at[pl.ds(r0 + k * _RC, _RC)])
        pltpu.sync_copy(zcb, cs_sh.at[pl.ds(r0 + k * _RC, _RC)])
    plsc.subcore_barrier()

    for c in range(_NCK):
        ef_hbm, tr_hbm, col_hbm = efs[c], trs[c], cols[c]

        def lstart(k, idx_b, ef_b, tr_b, sem):
            off = base + k * _C
            pltpu.async_copy(col_hbm.at[pl.ds(off, _C)], idx_b, sem)
            pltpu.async_copy(ef_hbm.at[pl.ds(off, _C)], ef_b, sem)
            pltpu.async_copy(tr_hbm.at[pl.ds(off, _C)], tr_b, sem)

        def lwait(idx_b, ef_b, tr_b, sem):
            pltpu.make_async_copy(col_hbm.at[pl.ds(0, _C)], idx_b, sem).wait()
            pltpu.make_async_copy(ef_hbm.at[pl.ds(0, _C)], ef_b, sem).wait()
            pltpu.make_async_copy(tr_hbm.at[pl.ds(0, _C)], tr_b, sem).wait()

        def scat(idx_b, ef_b, tr_b):
            pltpu.sync_copy(ef_b, agg_sh.at[idx_b], add=True)
            pltpu.sync_copy(tr_b, cs_sh.at[idx_b], add=True)

        A = (idxA, efA, trA)
        B = (idxB, efB, trB)
        lstart(0, *A, slA)

        def body(k2, carry):
            kA = 2 * k2
            kB = kA + 1

            @pl.when(kB < _NCHK)
            def _():
                lstart(kB, *B, slB)

            lwait(*A, slA)
            scat(*A)

            @pl.when(kA + 2 < _NCHK)
            def _():
                lstart(kA + 2, *A, slA)

            @pl.when(kB < _NCHK)
            def _():
                lwait(*B, slB)
                scat(*B)

            return carry

        lax.fori_loop(0, (_NCHK + 1) // 2, body, 0)

    plsc.subcore_barrier()
    for k in range(_NRC):
        rr = r0 + k * _RC
        pltpu.sync_copy(agg_sh.at[pl.ds(rr, _RC)], agg_o.at[cid, pl.ds(rr, _RC)])
        pltpu.sync_copy(cs_sh.at[pl.ds(rr, _RC)], cs_o.at[cid, pl.ds(rr, _RC)])


def _scatter(efs, trs, cols):
    mesh = plsc.VectorSubcoreMesh(core_axis_name="c", subcore_axis_name="s",
                                  num_cores=_NC, num_subcores=_NS)
    out_type = (jax.ShapeDtypeStruct((_NC, _N, _D), _f32),
                jax.ShapeDtypeStruct((_NC, _N, _CP), _f32))
    scratch = [pltpu.VMEM((_C,), jnp.int32),
               pltpu.VMEM((_C, _D), _f32),
               pltpu.VMEM((_C, _CP), _f32),
               pltpu.VMEM((_C,), jnp.int32),
               pltpu.VMEM((_C, _D), _f32),
               pltpu.VMEM((_C, _CP), _f32),
               pltpu.VMEM((_RC, _D), _f32),
               pltpu.VMEM((_RC, _CP), _f32),
               pltpu.VMEM_SHARED((_N, _D), _f32),
               pltpu.VMEM_SHARED((_N, _CP), _f32),
               pltpu.SemaphoreType.DMA, pltpu.SemaphoreType.DMA]
    return pl.kernel(_scatter_body, out_type=out_type, mesh=mesh,
                     scratch_types=scratch,
                     compiler_params=pltpu.CompilerParams(
                         use_tc_tiling_on_sc=False))(*efs, *trs, *cols)


_bf16 = jnp.bfloat16


def _edge_body(hr, hc, cr, cc, ea,
               W1ab, w1r, W1e, b1, W2, b2,
               cW1, cb1, cw2,
               eW1a, ew1r, eW1e, eb1, eW2, eb2,
               ef_o, tr_o, eo_o):
    cd = cr[...] - cc[...]
    radial = jnp.sum(cd * cd, axis=1, keepdims=True)
    norm = jnp.sqrt(radial + 1e-8)
    cdn = cd / (norm + 1.0)
    xa = ea[...]
    hcat = jnp.concatenate([hr[...], hc[...]], axis=1)
    y = (jnp.dot(hcat, W1ab[...], preferred_element_type=_f32)
         + radial * w1r[...][None, :]
         + jnp.dot(xa, W1e[...], preferred_element_type=_f32)
         + b1[...][None, :])
    ef = _silu(y)
    ef = _silu(jnp.dot(ef, W2[...], preferred_element_type=_f32)
               + b2[...][None, :])
    ef_o[...] = ef
    t = _silu(jnp.dot(ef, cW1[...], preferred_element_type=_f32)
              + cb1[...][None, :])
    coef = jnp.sum(t * cw2[...][None, :], axis=1, keepdims=True)
    tr_o[...] = cdn * coef
    e1 = _silu(jnp.dot(ef, eW1a[...], preferred_element_type=_f32)
               + radial * ew1r[...][None, :]
               + jnp.dot(xa, eW1e[...], preferred_element_type=_f32)
               + eb1[...][None, :])
    eo_o[...] = (jnp.dot(e1, eW2[...], preferred_element_type=_f32)
                 + eb2[...][None, :])


def _w_spec(shape):
    return pl.BlockSpec(shape, lambda i: tuple(0 for _ in shape))


_EDGE_KW = dict(
    grid=(_GE,),
    in_specs=[
        pl.BlockSpec((_TE, _D), lambda i: (i, 0)),
        pl.BlockSpec((_TE, _D), lambda i: (i, 0)),
        pl.BlockSpec((_TE, _CP), lambda i: (i, 0)),
        pl.BlockSpec((_TE, _CP), lambda i: (i, 0)),
        pl.BlockSpec((_TE, _CP), lambda i: (i, 0)),
        _w_spec((2 * _D, _D)), _w_spec((_D,)),
        _w_spec((_CP, _D)), _w_spec((_D,)), _w_spec((_D, _D)), _w_spec((_D,)),
        _w_spec((_D, _D)), _w_spec((_D,)), _w_spec((_D,)),
        _w_spec((_D, _D)), _w_spec((_D,)), _w_spec((_CP, _D)), _w_spec((_D,)),
        _w_spec((_D, _D)), _w_spec((_D,)),
    ],
    out_specs=[
        pl.BlockSpec((_TE, _D), lambda i: (i, 0)),
        pl.BlockSpec((_TE, _CP), lambda i: (i, 0)),
        pl.BlockSpec((_TE, _D), lambda i: (i, 0)),
    ],
    out_shape=[
        jax.ShapeDtypeStruct((_ECK, _D), _f32),
        jax.ShapeDtypeStruct((_ECK, _CP), _f32),
        jax.ShapeDtypeStruct((_ECK, _D), _f32),
    ],
)


def _node_body(h, aggp, csp, cp,
               nW1ab, nb1, nW2, nb2,
               ho_o, co_o):
    agg = aggp[0] + aggp[1]
    xh = h[...]
    hcat = jnp.concatenate([xh, agg], axis=1)
    y = _silu(jnp.dot(hcat, nW1ab[...], preferred_element_type=_f32)
              + nb1[...][None, :])
    ho_o[...] = (xh + jnp.dot(y, nW2[...], preferred_element_type=_f32)
                 + nb2[...][None, :])
    co_o[...] = cp[...] + csp[0] + csp[1]


_NODE_KW = dict(
    grid=(_GN,),
    in_specs=[
        pl.BlockSpec((_TN, _D), lambda i: (i, 0)),
        pl.BlockSpec((_NC, _TN, _D), lambda i: (0, i, 0)),
        pl.BlockSpec((_NC, _TN, _CP), lambda i: (0, i, 0)),
        pl.BlockSpec((_TN, _CP), lambda i: (i, 0)),
        _w_spec((2 * _D, _D)), _w_spec((_D,)),
        _w_spec((_D, _D)), _w_spec((_D,)),
    ],
    out_specs=[
        pl.BlockSpec((_TN, _D), lambda i: (i, 0)),
        pl.BlockSpec((_TN, _CP), lambda i: (i, 0)),
    ],
    out_shape=[
        jax.ShapeDtypeStruct((_N, _D), _f32),
        jax.ShapeDtypeStruct((_N, _CP), _f32),
    ],
)


def kernel(h, edge_index, coord, edge_attr,
           mes_W1, mes_b1, mes_W2, mes_b2,
           edge_W1, edge_b1, edge_W2, edge_b2,
           node_W1, node_b1, node_W2, node_b2,
           coord_W1, coord_b1, coord_W2):
    row = edge_index[0]
    col = edge_index[1]
    cp = jnp.pad(coord, ((0, 0), (0, _CP - 3)))
    eap = jnp.pad(edge_attr, ((0, 0), (0, 0)))

    efs, trs, eos = [], [], []
    cols = [col[c * _ECK:(c + 1) * _ECK] for c in range(_NCK)]
    rows = [row[c * _ECK:(c + 1) * _ECK] for c in range(_NCK)]
    for c in range(_NCK):
        hrow, hcol, crow, ccol = _gather(h, cp, rows[c], cols[c])
        ef_c, tr_c, eo_c = pl.pallas_call(_edge_body, **_EDGE_KW)(
            hrow, hcol, crow, ccol, eap[c * _ECK:(c + 1) * _ECK],
            mes_W1[:2 * _D], mes_W1[2 * _D], mes_W1[2 * _D + 1:],
            mes_b1, mes_W2, mes_b2,
            coord_W1, coord_b1, coord_W2[:, 0],
            edge_W1[:_D], edge_W1[_D], edge_W1[_D + 1:],
            edge_b1, edge_W2, edge_b2)
        efs.append(ef_c)
        trs.append(tr_c)
        eos.append(eo_c)

    edge_out = jnp.concatenate(eos, axis=0)
    aggp, csump = _scatter(efs, trs, cols)

    h_out, c16 = pl.pallas_call(_node_body, **_NODE_KW)(
        h, aggp, csump, cp,
        node_W1, node_b1, node_W2, node_b2)

    return (h_out, c16[:, :3], edge_out)


# speedup vs baseline: 1.0630x; 1.0630x over previous
import functools

import jax
import jax.numpy as jnp
from jax import lax
from jax.experimental import pallas as pl
from jax.experimental.pallas import tpu as pltpu
from jax.experimental.pallas import tpu_sc as plsc

_N = 10000
_E = 320000
_D = 128
_CP = 16

_NC = 2
_NS = 16
_NW = _NC * _NS
_EW = _E // _NW
_C = 80
_NCH = _EW // _C

_RW = _N // _NS
_RC = 25
_NRC = _RW // _RC

_TE = 4000
_GE = _E // _TE
_TN = 1000
_GN = _N // _TN

_f32 = jnp.float32


def _silu(x):
    return x * (1.0 / (1.0 + jnp.exp(-x)))


def _gather_body(h_hbm, cp_hbm, row_hbm, col_hbm,
                 hr_o, hc_o, cr_o, cc_o,
                 ir, ic,
                 hrA, hcA, crA, ccA, hrB, hcB, crB, ccB,
                 sgA, sgB, swA, swB):
    cid = lax.axis_index("c")
    sid = lax.axis_index("s")
    base = (sid * _NC + cid) * _EW

    pltpu.sync_copy(row_hbm.at[pl.ds(base, _EW)], ir)
    pltpu.sync_copy(col_hbm.at[pl.ds(base, _EW)], ic)

    def gstart(k, hr_b, hc_b, cr_b, cc_b, sem):
        s = pl.ds(k * _C, _C)
        pltpu.async_copy(h_hbm.at[ir.at[s]], hr_b, sem)
        pltpu.async_copy(h_hbm.at[ic.at[s]], hc_b, sem)
        pltpu.async_copy(cp_hbm.at[ir.at[s]], cr_b, sem)
        pltpu.async_copy(cp_hbm.at[ic.at[s]], cc_b, sem)

    def gwait(hr_b, hc_b, cr_b, cc_b, sem):
        pltpu.make_async_copy(h_hbm.at[pl.ds(0, _C)], hr_b, sem).wait()
        pltpu.make_async_copy(h_hbm.at[pl.ds(0, _C)], hc_b, sem).wait()
        pltpu.make_async_copy(cp_hbm.at[pl.ds(0, _C)], cr_b, sem).wait()
        pltpu.make_async_copy(cp_hbm.at[pl.ds(0, _C)], cc_b, sem).wait()

    def wstart(k, hr_b, hc_b, cr_b, cc_b, sem):
        off = base + k * _C
        pltpu.async_copy(hr_b, hr_o.at[pl.ds(off, _C)], sem)
        pltpu.async_copy(hc_b, hc_o.at[pl.ds(off, _C)], sem)
        pltpu.async_copy(cr_b, cr_o.at[pl.ds(off, _C)], sem)
        pltpu.async_copy(cc_b, cc_o.at[pl.ds(off, _C)], sem)

    def wwait(hr_b, hc_b, cr_b, cc_b, sem):
        pltpu.make_async_copy(hr_b, hr_o.at[pl.ds(0, _C)], sem).wait()
        pltpu.make_async_copy(hc_b, hc_o.at[pl.ds(0, _C)], sem).wait()
        pltpu.make_async_copy(cr_b, cr_o.at[pl.ds(0, _C)], sem).wait()
        pltpu.make_async_copy(cc_b, cc_o.at[pl.ds(0, _C)], sem).wait()

    A = (hrA, hcA, crA, ccA)
    B = (hrB, hcB, crB, ccB)
    gstart(0, *A, sgA)

    def body(k2, carry):
        kA = 2 * k2
        kB = kA + 1

        @pl.when(kB < _NCH)
        def _():
            gstart(kB, *B, sgB)

        gwait(*A, sgA)
        wstart(kA, *A, swA)

        @pl.when(kB < _NCH)
        def _():
            gwait(*B, sgB)
            wstart(kB, *B, swB)

        wwait(*A, swA)

        @pl.when(kA + 2 < _NCH)
        def _():
            gstart(kA + 2, *A, sgA)

        @pl.when(kB < _NCH)
        def _():
            wwait(*B, swB)

        return carry

    lax.fori_loop(0, (_NCH + 1) // 2, body, 0)


def _gather(h, cp, row, col):
    mesh = plsc.VectorSubcoreMesh(core_axis_name="c", subcore_axis_name="s",
                                  num_cores=_NC, num_subcores=_NS)
    hdt = h.dtype
    out_type = (jax.ShapeDtypeStruct((_E, _D), hdt),
                jax.ShapeDtypeStruct((_E, _D), hdt),
                jax.ShapeDtypeStruct((_E, _CP), _f32),
                jax.ShapeDtypeStruct((_E, _CP), _f32))
    scratch = [pltpu.VMEM((_EW,), jnp.int32),
               pltpu.VMEM((_EW,), jnp.int32),
               pltpu.VMEM((_C, _D), hdt), pltpu.VMEM((_C, _D), hdt),
               pltpu.VMEM((_C, _CP), _f32), pltpu.VMEM((_C, _CP), _f32),
               pltpu.VMEM((_C, _D), hdt), pltpu.VMEM((_C, _D), hdt),
               pltpu.VMEM((_C, _CP), _f32), pltpu.VMEM((_C, _CP), _f32),
               pltpu.SemaphoreType.DMA, pltpu.SemaphoreType.DMA,
               pltpu.SemaphoreType.DMA, pltpu.SemaphoreType.DMA]
    return pl.kernel(_gather_body, out_type=out_type, mesh=mesh,
                     scratch_types=scratch,
                     compiler_params=pltpu.CompilerParams(
                         use_tc_tiling_on_sc=False))(h, cp, row, col)


def _scatter_body(ef_hbm, tr_hbm, col_hbm,
                  agg_o, cs_o,
                  idxA, efA, trA, idxB, efB, trB,
                  zbuf, zcb, agg_sh, cs_sh, slA, slB):
    cid = lax.axis_index("c")
    sid = lax.axis_index("s")
    base = (sid * _NC + cid) * _EW
    z16 = jnp.zeros((16,), _f32)

    def zrow(i, carry):
        for j in range(_D // 16):
            zbuf[i, pl.ds(j * 16, 16)] = z16
        zcb[i, :] = z16
        return carry

    lax.fori_loop(0, _RC, zrow, 0)
    r0 = sid * _RW
    for k in range(_NRC):
        pltpu.sync_copy(zbuf, agg_sh.at[pl.ds(r0 + k * _RC, _RC)])
        pltpu.sync_copy(zcb, cs_sh.at[pl.ds(r0 + k * _RC, _RC)])
    plsc.subcore_barrier()

    def lstart(k, idx_b, ef_b, tr_b, sem):
        off = base + k * _C
        pltpu.async_copy(col_hbm.at[pl.ds(off, _C)], idx_b, sem)
        pltpu.async_copy(ef_hbm.at[pl.ds(off, _C)], ef_b, sem)
        pltpu.async_copy(tr_hbm.at[pl.ds(off, _C)], tr_b, sem)

    def lwait(idx_b, ef_b, tr_b, sem):
        pltpu.make_async_copy(col_hbm.at[pl.ds(0, _C)], idx_b, sem).wait()
        pltpu.make_async_copy(ef_hbm.at[pl.ds(0, _C)], ef_b, sem).wait()
        pltpu.make_async_copy(tr_hbm.at[pl.ds(0, _C)], tr_b, sem).wait()


    def scat(idx_b, ef_b, tr_b):
        pltpu.sync_copy(ef_b, agg_sh.at[idx_b], add=True)
        pltpu.sync_copy(tr_b, cs_sh.at[idx_b], add=True)

    A = (idxA, efA, trA)
    B = (idxB, efB, trB)
    lstart(0, *A, slA)

    def body(k2, carry):
        kA = 2 * k2
        kB = kA + 1

        @pl.when(kB < _NCH)
        def _():
            lstart(kB, *B, slB)

        lwait(*A, slA)
        scat(*A)

        @pl.when(kA + 2 < _NCH)
        def _():
            lstart(kA + 2, *A, slA)

        @pl.when(kB < _NCH)
        def _():
            lwait(*B, slB)
            scat(*B)

        return carry

    lax.fori_loop(0, (_NCH + 1) // 2, body, 0)
    plsc.subcore_barrier()
    for k in range(_NRC):
        rr = r0 + k * _RC
        pltpu.sync_copy(agg_sh.at[pl.ds(rr, _RC)], agg_o.at[cid, pl.ds(rr, _RC)])
        pltpu.sync_copy(cs_sh.at[pl.ds(rr, _RC)], cs_o.at[cid, pl.ds(rr, _RC)])


def _scatter(ef, tr, col):
    mesh = plsc.VectorSubcoreMesh(core_axis_name="c", subcore_axis_name="s",
                                  num_cores=_NC, num_subcores=_NS)
    out_type = (jax.ShapeDtypeStruct((_NC, _N, _D), _f32),
                jax.ShapeDtypeStruct((_NC, _N, _CP), _f32))
    scratch = [pltpu.VMEM((_C,), jnp.int32),
               pltpu.VMEM((_C, _D), _f32),
               pltpu.VMEM((_C, _CP), _f32),
               pltpu.VMEM((_C,), jnp.int32),
               pltpu.VMEM((_C, _D), _f32),
               pltpu.VMEM((_C, _CP), _f32),
               pltpu.VMEM((_RC, _D), _f32),
               pltpu.VMEM((_RC, _CP), _f32),
               pltpu.VMEM_SHARED((_N, _D), _f32),
               pltpu.VMEM_SHARED((_N, _CP), _f32),
               pltpu.SemaphoreType.DMA, pltpu.SemaphoreType.DMA]
    return pl.kernel(_scatter_body, out_type=out_type, mesh=mesh,
                     scratch_types=scratch,
                     compiler_params=pltpu.CompilerParams(
                         use_tc_tiling_on_sc=False))(ef, tr, col)


_bf16 = jnp.bfloat16


def _edge_body(hr, hc, cr, cc, ea,
               W1ab, w1r, W1e, b1, W2, b2,
               cW1, cb1, cw2,
               eW1a, ew1r, eW1e, eb1, eW2, eb2,
               ef_o, tr_o, eo_o):
    cd = cr[...] - cc[...]
    radial = jnp.sum(cd * cd, axis=1, keepdims=True)
    norm = jnp.sqrt(radial + 1e-8)
    cdn = cd / (norm + 1.0)
    xa = ea[...]
    hcat = jnp.concatenate([hr[...], hc[...]], axis=1)
    y = (jnp.dot(hcat, W1ab[...], preferred_element_type=_f32)
         + radial * w1r[...][None, :]
         + jnp.dot(xa, W1e[...], preferred_element_type=_f32)
         + b1[...][None, :])
    ef = _silu(y)
    ef = _silu(jnp.dot(ef, W2[...], preferred_element_type=_f32)
               + b2[...][None, :])
    ef_o[...] = ef
    t = _silu(jnp.dot(ef, cW1[...], preferred_element_type=_f32)
              + cb1[...][None, :])
    coef = jnp.sum(t * cw2[...][None, :], axis=1, keepdims=True)
    tr_o[...] = cdn * coef
    e1 = _silu(jnp.dot(ef, eW1a[...], preferred_element_type=_f32)
               + radial * ew1r[...][None, :]
               + jnp.dot(xa, eW1e[...], preferred_element_type=_f32)
               + eb1[...][None, :])
    eo_o[...] = (jnp.dot(e1, eW2[...], preferred_element_type=_f32)
                 + eb2[...][None, :])


def _w_spec(shape):
    return pl.BlockSpec(shape, lambda i: tuple(0 for _ in shape))


_EDGE_KW = dict(
    grid=(_GE,),
    in_specs=[
        pl.BlockSpec((_TE, _D), lambda i: (i, 0)),
        pl.BlockSpec((_TE, _D), lambda i: (i, 0)),
        pl.BlockSpec((_TE, _CP), lambda i: (i, 0)),
        pl.BlockSpec((_TE, _CP), lambda i: (i, 0)),
        pl.BlockSpec((_TE, _CP), lambda i: (i, 0)),
        _w_spec((2 * _D, _D)), _w_spec((_D,)),
        _w_spec((_CP, _D)), _w_spec((_D,)), _w_spec((_D, _D)), _w_spec((_D,)),
        _w_spec((_D, _D)), _w_spec((_D,)), _w_spec((_D,)),
        _w_spec((_D, _D)), _w_spec((_D,)), _w_spec((_CP, _D)), _w_spec((_D,)),
        _w_spec((_D, _D)), _w_spec((_D,)),
    ],
    out_specs=[
        pl.BlockSpec((_TE, _D), lambda i: (i, 0)),
        pl.BlockSpec((_TE, _CP), lambda i: (i, 0)),
        pl.BlockSpec((_TE, _D), lambda i: (i, 0)),
    ],
    out_shape=[
        jax.ShapeDtypeStruct((_E, _D), _f32),
        jax.ShapeDtypeStruct((_E, _CP), _f32),
        jax.ShapeDtypeStruct((_E, _D), _f32),
    ],
)


def _node_body(h, aggp, csp, cp,
               nW1ab, nb1, nW2, nb2,
               ho_o, co_o):
    agg = aggp[0] + aggp[1]
    xh = h[...]
    hcat = jnp.concatenate([xh, agg], axis=1)
    y = _silu(jnp.dot(hcat, nW1ab[...], preferred_element_type=_f32)
              + nb1[...][None, :])
    ho_o[...] = (xh + jnp.dot(y, nW2[...], preferred_element_type=_f32)
                 + nb2[...][None, :])
    co_o[...] = cp[...] + csp[0] + csp[1]


_NODE_KW = dict(
    grid=(_GN,),
    in_specs=[
        pl.BlockSpec((_TN, _D), lambda i: (i, 0)),
        pl.BlockSpec((_NC, _TN, _D), lambda i: (0, i, 0)),
        pl.BlockSpec((_NC, _TN, _CP), lambda i: (0, i, 0)),
        pl.BlockSpec((_TN, _CP), lambda i: (i, 0)),
        _w_spec((2 * _D, _D)), _w_spec((_D,)),
        _w_spec((_D, _D)), _w_spec((_D,)),
    ],
    out_specs=[
        pl.BlockSpec((_TN, _D), lambda i: (i, 0)),
        pl.BlockSpec((_TN, _CP), lambda i: (i, 0)),
    ],
    out_shape=[
        jax.ShapeDtypeStruct((_N, _D), _f32),
        jax.ShapeDtypeStruct((_N, _CP), _f32),
    ],
)


def kernel(h, edge_index, coord, edge_attr,
           mes_W1, mes_b1, mes_W2, mes_b2,
           edge_W1, edge_b1, edge_W2, edge_b2,
           node_W1, node_b1, node_W2, node_b2,
           coord_W1, coord_b1, coord_W2):
    row = edge_index[0]
    col = edge_index[1]
    cp = jnp.pad(coord, ((0, 0), (0, _CP - 3)))
    eap = jnp.pad(edge_attr, ((0, 0), (0, 0)))

    hrow, hcol, crow, ccol = _gather(h, cp, row, col)

    ef, trans, edge_out = pl.pallas_call(_edge_body, **_EDGE_KW)(
        hrow, hcol, crow, ccol, eap,
        mes_W1[:2 * _D], mes_W1[2 * _D], mes_W1[2 * _D + 1:],
        mes_b1, mes_W2, mes_b2,
        coord_W1, coord_b1, coord_W2[:, 0],
        edge_W1[:_D], edge_W1[_D], edge_W1[_D + 1:],
        edge_b1, edge_W2, edge_b2)

    aggp, csump = _scatter(ef, trans, col)

    h_out, c16 = pl.pallas_call(_node_body, **_NODE_KW)(
        h, aggp, csump, cp,
        node_W1, node_b1, node_W2, node_b2)

    return (h_out, c16[:, :3], edge_out)
